# Initial kernel scaffold; baseline (speedup 1.0000x reference)
import jax
import jax.numpy as jnp
from jax.experimental import pallas as pl


def kernel(input, table):
    raise NotImplementedError("write your pallas kernel here")


# speedup vs baseline: 2.4158x; 2.4158x over previous
import functools
import math

import jax
import jax.numpy as jnp
from jax import lax
from jax.experimental import pallas as pl
from jax.experimental.pallas import tpu as pltpu
from jax.experimental.pallas import tpu_sc as plsc

DIM = 128
SCALE = math.sqrt(float(DIM))

_NC = 2
_NS = 16
_NW = _NC * _NS

CHUNK = 128


@functools.lru_cache(maxsize=None)
def _make_kernel(n_idx):
    assert n_idx % (_NW * CHUNK) == 0
    chunks_per_w = n_idx // (_NW * CHUNK)
    mesh = plsc.VectorSubcoreMesh(core_axis_name="c", subcore_axis_name="s")

    @functools.partial(
        pl.kernel,
        mesh=mesh,
        out_type=jax.ShapeDtypeStruct((n_idx, DIM), jnp.float32),
        scratch_types=[
            pltpu.VMEM((chunks_per_w, CHUNK), jnp.int32),
            pltpu.VMEM((CHUNK, DIM), jnp.float32),
            pltpu.SemaphoreType.DMA,
        ],
    )
    def body(idx_hbm, table_hbm, out_hbm, idx_v, rows_v, sem):
        wid = lax.axis_index("s") * _NC + lax.axis_index("c")
        row0 = wid * chunks_per_w
        pltpu.sync_copy(idx_hbm.at[wid], idx_v)

        def chunk_body(j, carry):
            pltpu.async_copy(table_hbm.at[idx_v.at[j]], rows_v, sem).wait()

            def scale_body(i, c):
                for u in range(DIM // 16):
                    sl = pl.ds(u * 16, 16)
                    rows_v[i, sl] = rows_v[i, sl] * SCALE
                return c

            lax.fori_loop(0, CHUNK, scale_body, 0)
            out0 = (row0 + j) * CHUNK
            pltpu.sync_copy(rows_v, out_hbm.at[pl.ds(out0, CHUNK)])
            return carry

        lax.fori_loop(0, chunks_per_w, chunk_body, 0)

    return body


def kernel(input, table):
    b, h = input.shape
    idx2 = input.reshape(_NW, b * h // (_NW * CHUNK), CHUNK)
    out = _make_kernel(b * h)(idx2, table)
    return out.reshape(b, h, DIM)


# speedup vs baseline: 2.8229x; 1.1685x over previous
import functools
import math

import jax
import jax.numpy as jnp
from jax import lax
from jax.experimental import pallas as pl
from jax.experimental.pallas import tpu as pltpu
from jax.experimental.pallas import tpu_sc as plsc

DIM = 128
SCALE = math.sqrt(float(DIM))

_NC = 2
_NS = 16
_NW = _NC * _NS

CHUNK = 128


@functools.lru_cache(maxsize=None)
def _make_kernel(n_idx):
    assert n_idx % (_NW * CHUNK) == 0
    chunks_per_w = n_idx // (_NW * CHUNK)
    assert chunks_per_w % 2 == 0 and chunks_per_w >= 4
    mesh = plsc.VectorSubcoreMesh(core_axis_name="c", subcore_axis_name="s")

    @functools.partial(
        pl.kernel,
        mesh=mesh,
        out_type=jax.ShapeDtypeStruct((n_idx, DIM), jnp.float32),
        scratch_types=[
            pltpu.VMEM((chunks_per_w, CHUNK), jnp.int32),
            pltpu.VMEM((2, CHUNK, DIM), jnp.float32),
            pltpu.SemaphoreType.DMA,
            pltpu.SemaphoreType.DMA,
            pltpu.SemaphoreType.DMA,
            pltpu.SemaphoreType.DMA,
        ],
    )
    def body(idx_hbm, table_hbm, out_hbm, idx_v, rows_v, g0, g1, o0, o1):
        wid = lax.axis_index("s") * _NC + lax.axis_index("c")
        row0 = wid * chunks_per_w
        pltpu.sync_copy(idx_hbm.at[wid], idx_v)

        gsem = (g0, g1)
        osem = (o0, o1)

        def g_copy(j, b):
            return pltpu.make_async_copy(
                table_hbm.at[idx_v.at[j]], rows_v.at[b], gsem[b])

        def o_copy(j, b):
            return pltpu.make_async_copy(
                rows_v.at[b],
                out_hbm.at[pl.ds((row0 + j) * CHUNK, CHUNK)],
                osem[b])

        def scale(b):
            def sb(i, c):
                for u in range(DIM // 16):
                    sl = pl.ds(u * 16, 16)
                    rows_v[b, i, sl] = rows_v[b, i, sl] * SCALE
                return c
            lax.fori_loop(0, CHUNK, sb, 0)

        def step(j, b, first, last):
            g_copy(j, b).wait()
            if not first:
                o_copy(j - 1, 1 - b).wait()
            if not last:
                g_copy(j + 1, 1 - b).start()
            scale(b)
            o_copy(j, b).start()

        g_copy(0, 0).start()
        step(0, 0, True, False)
        step(1, 1, False, False)

        def loop_body(jp, c):
            step(2 * jp, 0, False, False)
            step(2 * jp + 1, 1, False, False)
            return c

        lax.fori_loop(1, chunks_per_w // 2 - 1, loop_body, 0)

        step(chunks_per_w - 2, 0, False, False)
        step(chunks_per_w - 1, 1, False, True)
        o_copy(chunks_per_w - 1, 1).wait()

    return body


def kernel(input, table):
    b, h = input.shape
    idx2 = input.reshape(_NW, b * h // (_NW * CHUNK), CHUNK)
    out = _make_kernel(b * h)(idx2, table)
    return out.reshape(b, h, DIM)
